# baseline (speedup 1.0000x reference)
import functools

import jax
import jax.numpy as jnp
from jax import lax
from jax.experimental import pallas as pl
from jax.experimental.pallas import tpu as pltpu
from jax.experimental.pallas import tpu_sc as plsc

NU = 5000
NN = 10000
NT = 2 * NN
DH = 128
NL = 4
NE = 160000
NS = 16
CH = 128
NCHT = NE // CH
RPT = 624

_mesh = plsc.VectorSubcoreMesh(core_axis_name="c", subcore_axis_name="s")


def _f32(shape):
    return jax.ShapeDtypeStruct(shape, jnp.float32)


@functools.partial(
    pl.kernel,
    out_type=[_f32((NT, DH)), _f32(((NL + 1) * NT, DH))],
    mesh=_mesh,
    scratch_types=[
        [pltpu.VMEM((CH, DH), jnp.float32) for _ in range(3)],
        [pltpu.VMEM((CH,), jnp.int32) for _ in range(3)],
        [pltpu.VMEM((CH,), jnp.int32) for _ in range(3)],
        [pltpu.VMEM((CH,), jnp.float32) for _ in range(3)],
        pltpu.VMEM_SHARED((NN, DH), jnp.float32),
        [pltpu.SemaphoreType.DMA for _ in range(3)],
        [pltpu.SemaphoreType.DMA for _ in range(3)],
        [pltpu.SemaphoreType.DMA for _ in range(3)],
        [pltpu.SemaphoreType.DMA for _ in range(3)],
    ],
)
def _lightgcn_sc(src_hbm, dst_hbm, w_hbm, x_hbm,
                 y_hbm, lf_hbm,
                 rows_v, sidx_v, didx_v, w_v, acc_sh,
                 sem_g, sem_s, sem_i, sem_d):
    c = lax.axis_index("c")
    s = lax.axis_index("s")
    coff = c * NN
    abase = s * RPT
    zv = jnp.zeros((16,), jnp.float32)
    nch = jnp.int32(NCHT // NS) + jnp.where(s < NCHT % NS, 1, 0)

    def sw_start(i, b):
        base = (s + NS * i) * CH
        pltpu.async_copy(src_hbm.at[pl.ds(base, CH)], sidx_v[b], sem_i[b])
        pltpu.async_copy(w_hbm.at[pl.ds(base, CH)], w_v[b], sem_i[b])

    def sw_wait(b):
        pltpu.make_async_copy(src_hbm.at[pl.ds(0, CH)], sidx_v[b],
                              sem_i[b]).wait()
        pltpu.make_async_copy(w_hbm.at[pl.ds(0, CH)], w_v[b],
                              sem_i[b]).wait()

    def didx_start(i, b):
        base = (s + NS * i) * CH
        pltpu.async_copy(dst_hbm.at[pl.ds(base, CH)], didx_v[b], sem_d[b])

    def didx_wait(b):
        pltpu.make_async_copy(dst_hbm.at[pl.ds(0, CH)], didx_v[b],
                              sem_d[b]).wait()

    def gather_start(roff, b):
        @plsc.parallel_loop(0, CH // 16, unroll=2)
        def rebase(j):
            sl = pl.ds(j * 16, 16)
            sidx_v[b][sl] = sidx_v[b][sl] + roff

        pltpu.async_copy(lf_hbm.at[sidx_v[b]], rows_v[b], sem_g[b])

    def gather_wait(b):
        pltpu.make_async_copy(lf_hbm.at[sidx_v[b]], rows_v[b],
                              sem_g[b]).wait()

    def scatter_start(b):
        pltpu.async_copy(rows_v[b], acc_sh.at[didx_v[b]], sem_s[b], add=True)

    def scatter_wait(b):
        pltpu.make_async_copy(rows_v[b], acc_sh.at[didx_v[b]],
                              sem_s[b]).wait()

    def scale(b):
        @plsc.parallel_loop(0, CH // 16, unroll=4)
        def scale_grp(g):
            wgrp = w_v[b][pl.ds(g * 16, 16)]
            for lane in range(16):
                wb = wgrp.at[jnp.full((16,), lane, jnp.int32)].get(
                    mode="promise_in_bounds")
                r = g * 16 + lane
                for k in range(DH // 16):
                    sl = pl.ds(k * 16, 16)
                    rows_v[b][r, sl] = rows_v[b][r, sl] * wb

    pltpu.sync_copy(x_hbm.at[pl.ds(coff + abase, RPT)],
                    lf_hbm.at[pl.ds(coff + abase, RPT)])

    @pl.when(s == NS - 1)
    def _():
        pltpu.sync_copy(x_hbm.at[pl.ds(coff + NS * RPT, 16)],
                        lf_hbm.at[pl.ds(coff + NS * RPT, 16)])

    plsc.subcore_barrier()

    def layer(l, carry):
        roff = l * NT + coff

        @plsc.parallel_loop(0, CH, unroll=2)
        def zrow(r):
            for k in range(DH // 16):
                rows_v[0][r, pl.ds(k * 16, 16)] = zv

        for off, sz in ((0, 128), (128, 128), (256, 128), (384, 128),
                        (512, 112)):
            pltpu.sync_copy(rows_v[0].at[pl.ds(0, sz)],
                            acc_sh.at[pl.ds(abase + off, sz)])

        @pl.when(s == NS - 1)
        def _():
            pltpu.sync_copy(rows_v[0].at[pl.ds(0, 16)],
                            acc_sh.at[pl.ds(NS * RPT, 16)])

        plsc.subcore_barrier()

        for i0, b0 in ((0, 0), (1, 1)):
            sw_start(i0, b0)
            didx_start(i0, b0)
            sw_wait(b0)
            gather_start(roff, b0)

        def process(i, b):
            b2 = (b + 2) % 3

            @pl.when(i + 2 < nch)
            def _():
                sw_start(i + 2, b2)

            gather_wait(b)
            scale(b)
            didx_wait(b)
            scatter_start(b)

            @pl.when(i + 2 < nch)
            def _():
                @pl.when(i >= 1)
                def _():
                    scatter_wait(b2)

                didx_start(i + 2, b2)
                sw_wait(b2)
                gather_start(roff, b2)

        def pipe_iter(j, cy):
            for d, b in ((0, 0), (1, 1), (2, 2)):
                i = 3 * j + d

                @pl.when(i < nch)
                def _():
                    process(i, b)

            return cy

        lax.fori_loop(0, (nch + 2) // 3, pipe_iter, 0)
        for b in range(3):
            scatter_wait(b)
        plsc.subcore_barrier()

        obase = (l + 1) * NT + coff
        pltpu.sync_copy(acc_sh.at[pl.ds(abase, RPT)],
                        lf_hbm.at[pl.ds(obase + abase, RPT)])

        @pl.when(s == NS - 1)
        def _():
            pltpu.sync_copy(acc_sh.at[pl.ds(NS * RPT, 16)],
                            lf_hbm.at[pl.ds(obase + NS * RPT, 16)])

        plsc.subcore_barrier()
        return carry

    lax.fori_loop(0, NL, layer, 0)

    def mean_chunk(mbase, sz):
        pltpu.sync_copy(lf_hbm.at[pl.ds(mbase, sz)],
                        rows_v[0].at[pl.ds(0, sz)])

        def macc(l, cy):
            pltpu.sync_copy(lf_hbm.at[pl.ds(l * NT + mbase, sz)],
                            rows_v[1].at[pl.ds(0, sz)])

            @plsc.parallel_loop(0, sz, unroll=2)
            def madd(r):
                for k in range(DH // 16):
                    sl = pl.ds(k * 16, 16)
                    rows_v[0][r, sl] = rows_v[0][r, sl] + rows_v[1][r, sl]

            return cy

        lax.fori_loop(1, NL + 1, macc, 0)

        @plsc.parallel_loop(0, sz, unroll=2)
        def mfin(r):
            for k in range(DH // 16):
                sl = pl.ds(k * 16, 16)
                rows_v[0][r, sl] = rows_v[0][r, sl] * jnp.float32(1.0 / (NL + 1))

        pltpu.sync_copy(rows_v[0].at[pl.ds(0, sz)],
                        y_hbm.at[pl.ds(mbase, sz)])

    for off, sz in ((0, 128), (128, 128), (256, 128), (384, 128),
                    (512, 112)):
        mean_chunk(coff + abase + off, sz)

    @pl.when(s == NS - 1)
    def _():
        mean_chunk(coff + NS * RPT, 16)


def kernel(edge_index, edge_weight, user_emb, item_emb):
    src = edge_index[0].astype(jnp.int32)
    dst = edge_index[1].astype(jnp.int32)
    all_emb = jnp.concatenate([user_emb, item_emb], axis=0)
    x2 = jnp.concatenate([all_emb[:, :DH], all_emb[:, DH:]], axis=0)
    y = _lightgcn_sc(src, dst, edge_weight, x2)[0]
    final = jnp.concatenate([y[:NN], y[NN:]], axis=1)
    return (final[:NU], final[NU:])


# speedup vs baseline: 1.0539x; 1.0539x over previous
import functools

import jax
import jax.numpy as jnp
from jax import lax
from jax.experimental import pallas as pl
from jax.experimental.pallas import tpu as pltpu
from jax.experimental.pallas import tpu_sc as plsc

NU = 5000
NN = 10000
NT = 2 * NN
DH = 128
NL = 4
NE = 160000
NS = 16
CH = 128
NCHT = NE // CH
RPT = 624

_mesh = plsc.VectorSubcoreMesh(core_axis_name="c", subcore_axis_name="s")


def _f32(shape):
    return jax.ShapeDtypeStruct(shape, jnp.float32)


@functools.partial(
    pl.kernel,
    out_type=[_f32((NT, DH)), _f32(((NL + 1) * NT, DH))],
    mesh=_mesh,
    scratch_types=[
        [pltpu.VMEM((CH, DH), jnp.float32) for _ in range(3)],
        [pltpu.VMEM((CH,), jnp.int32) for _ in range(3)],
        [pltpu.VMEM((CH,), jnp.int32) for _ in range(3)],
        [pltpu.VMEM((CH,), jnp.float32) for _ in range(3)],
        pltpu.VMEM_SHARED((NN, DH), jnp.float32),
        [pltpu.SemaphoreType.DMA for _ in range(3)],
        [pltpu.SemaphoreType.DMA for _ in range(3)],
        [pltpu.SemaphoreType.DMA for _ in range(3)],
        [pltpu.SemaphoreType.DMA for _ in range(3)],
    ],
)
def _lightgcn_sc(src_hbm, dst_hbm, w_hbm, x_hbm,
                 y_hbm, lf_hbm,
                 rows_v, sidx_v, didx_v, w_v, acc_sh,
                 sem_g, sem_s, sem_i, sem_d):
    c = lax.axis_index("c")
    s = lax.axis_index("s")
    coff = c * NN
    abase = s * RPT
    zv = jnp.zeros((16,), jnp.float32)
    nch = jnp.int32(NCHT // NS) + jnp.where(s < NCHT % NS, 1, 0)

    def sw_start(i, b):
        base = (s + NS * i) * CH
        pltpu.async_copy(src_hbm.at[pl.ds(base, CH)], sidx_v[b], sem_i[b])
        pltpu.async_copy(w_hbm.at[pl.ds(base, CH)], w_v[b], sem_i[b])

    def sw_wait(b):
        pltpu.make_async_copy(src_hbm.at[pl.ds(0, CH)], sidx_v[b],
                              sem_i[b]).wait()
        pltpu.make_async_copy(w_hbm.at[pl.ds(0, CH)], w_v[b],
                              sem_i[b]).wait()

    def didx_start(i, b):
        base = (s + NS * i) * CH
        pltpu.async_copy(dst_hbm.at[pl.ds(base, CH)], didx_v[b], sem_d[b])

    def didx_wait(b):
        pltpu.make_async_copy(dst_hbm.at[pl.ds(0, CH)], didx_v[b],
                              sem_d[b]).wait()

    def gather_start(roff, b):
        @plsc.parallel_loop(0, CH // 16, unroll=2)
        def rebase(j):
            sl = pl.ds(j * 16, 16)
            sidx_v[b][sl] = sidx_v[b][sl] + roff

        pltpu.async_copy(lf_hbm.at[sidx_v[b]], rows_v[b], sem_g[b])

    def gather_wait(b):
        pltpu.make_async_copy(lf_hbm.at[sidx_v[b]], rows_v[b],
                              sem_g[b]).wait()

    def scatter_start(b):
        pltpu.async_copy(rows_v[b], acc_sh.at[didx_v[b]], sem_s[b], add=True)

    def scatter_wait(b):
        pltpu.make_async_copy(rows_v[b], acc_sh.at[didx_v[b]],
                              sem_s[b]).wait()

    def scale(b):
        @plsc.parallel_loop(0, CH // 16, unroll=2)
        def scale_grp(g):
            wgrp = w_v[b][pl.ds(g * 16, 16)]
            for lane in range(16):
                wb = wgrp.at[jnp.full((16,), lane, jnp.int32)].get(
                    mode="promise_in_bounds")
                r = g * 16 + lane
                for k in range(DH // 16):
                    sl = pl.ds(k * 16, 16)
                    rows_v[b][r, sl] = rows_v[b][r, sl] * wb

    pltpu.sync_copy(x_hbm.at[pl.ds(coff + abase, RPT)],
                    lf_hbm.at[pl.ds(coff + abase, RPT)])

    @pl.when(s == NS - 1)
    def _():
        pltpu.sync_copy(x_hbm.at[pl.ds(coff + NS * RPT, 16)],
                        lf_hbm.at[pl.ds(coff + NS * RPT, 16)])

    plsc.subcore_barrier()

    def layer(l, carry):
        roff = l * NT + coff

        @plsc.parallel_loop(0, CH, unroll=2)
        def zrow(r):
            for k in range(DH // 16):
                rows_v[0][r, pl.ds(k * 16, 16)] = zv

        for off, sz in ((0, 128), (128, 128), (256, 128), (384, 128),
                        (512, 112)):
            pltpu.sync_copy(rows_v[0].at[pl.ds(0, sz)],
                            acc_sh.at[pl.ds(abase + off, sz)])

        @pl.when(s == NS - 1)
        def _():
            pltpu.sync_copy(rows_v[0].at[pl.ds(0, 16)],
                            acc_sh.at[pl.ds(NS * RPT, 16)])

        plsc.subcore_barrier()

        for i0, b0 in ((0, 0), (1, 1)):
            sw_start(i0, b0)
            didx_start(i0, b0)
            sw_wait(b0)
            gather_start(roff, b0)

        def process(i, b):
            b2 = (b + 2) % 3

            @pl.when(i + 2 < nch)
            def _():
                sw_start(i + 2, b2)

            gather_wait(b)
            scale(b)
            didx_wait(b)
            scatter_start(b)

            @pl.when(i + 2 < nch)
            def _():
                @pl.when(i >= 1)
                def _():
                    scatter_wait(b2)

                didx_start(i + 2, b2)
                sw_wait(b2)
                gather_start(roff, b2)

        def pipe_iter(j, cy):
            for d, b in ((0, 0), (1, 1), (2, 2)):
                i = 3 * j + d

                @pl.when(i < nch)
                def _():
                    process(i, b)

            return cy

        lax.fori_loop(0, (nch + 2) // 3, pipe_iter, 0)
        for b in range(3):
            scatter_wait(b)
        plsc.subcore_barrier()

        obase = (l + 1) * NT + coff
        pltpu.sync_copy(acc_sh.at[pl.ds(abase, RPT)],
                        lf_hbm.at[pl.ds(obase + abase, RPT)])

        @pl.when(s == NS - 1)
        def _():
            pltpu.sync_copy(acc_sh.at[pl.ds(NS * RPT, 16)],
                            lf_hbm.at[pl.ds(obase + NS * RPT, 16)])

        plsc.subcore_barrier()
        return carry

    lax.fori_loop(0, NL, layer, 0)

    def mean_chunk(mbase, sz):
        pltpu.sync_copy(lf_hbm.at[pl.ds(mbase, sz)],
                        rows_v[0].at[pl.ds(0, sz)])

        def macc(l, cy):
            pltpu.sync_copy(lf_hbm.at[pl.ds(l * NT + mbase, sz)],
                            rows_v[1].at[pl.ds(0, sz)])

            @plsc.parallel_loop(0, sz, unroll=2)
            def madd(r):
                for k in range(DH // 16):
                    sl = pl.ds(k * 16, 16)
                    rows_v[0][r, sl] = rows_v[0][r, sl] + rows_v[1][r, sl]

            return cy

        lax.fori_loop(1, NL + 1, macc, 0)

        @plsc.parallel_loop(0, sz, unroll=2)
        def mfin(r):
            for k in range(DH // 16):
                sl = pl.ds(k * 16, 16)
                rows_v[0][r, sl] = rows_v[0][r, sl] * jnp.float32(1.0 / (NL + 1))

        pltpu.sync_copy(rows_v[0].at[pl.ds(0, sz)],
                        y_hbm.at[pl.ds(mbase, sz)])

    for off, sz in ((0, 128), (128, 128), (256, 128), (384, 128),
                    (512, 112)):
        mean_chunk(coff + abase + off, sz)

    @pl.when(s == NS - 1)
    def _():
        mean_chunk(coff + NS * RPT, 16)


def kernel(edge_index, edge_weight, user_emb, item_emb):
    src = edge_index[0].astype(jnp.int32)
    dst = edge_index[1].astype(jnp.int32)
    all_emb = jnp.concatenate([user_emb, item_emb], axis=0)
    x2 = jnp.concatenate([all_emb[:, :DH], all_emb[:, DH:]], axis=0)
    y = _lightgcn_sc(src, dst, edge_weight, x2)[0]
    final = jnp.concatenate([y[:NN], y[NN:]], axis=1)
    return (final[:NU], final[NU:])
